# Initial kernel scaffold; baseline (speedup 1.0000x reference)
import jax
import jax.numpy as jnp
from jax.experimental import pallas as pl


def kernel(unknown, known, unknow_feats, known_feats, W0, gamma0, beta0):
    raise NotImplementedError("write your pallas kernel here")


# speedup vs baseline: 18.7885x; 18.7885x over previous
import jax
import jax.numpy as jnp
from jax.experimental import pallas as pl

_B, _N, _M, _C1, _C2 = 4, 8192, 1024, 64, 128
_COUT = 128
_NT = 256
_NB = _N // _NT


def _fp_fwd_kernel(uT_ref, known_ref, uf_ref, kf_ref, w0_ref,
                   x_ref, sum_ref, ssq_ref):
    uT = uT_ref[0]
    kn = known_ref[0]

    cross = jax.lax.dot(kn.astype(jnp.bfloat16), uT.astype(jnp.bfloat16),
                        preferred_element_type=jnp.float32)
    k2 = jnp.sum(kn * kn, axis=1, keepdims=True)
    u2 = jnp.sum(uT * uT, axis=0, keepdims=True)
    d2 = (u2 + k2) - 2.0 * cross

    iota = jax.lax.broadcasted_iota(jnp.int32, (_M, _NT), 0)
    inf = jnp.float32(jnp.inf)
    work = d2
    vs = []
    ix = []
    for _ in range(3):
        v = jnp.min(work, axis=0, keepdims=True)
        i = jnp.min(jnp.where(work == v, iota, _M), axis=0, keepdims=True)
        vs.append(v)
        ix.append(i)
        work = jnp.where(iota == i, inf, work)

    r = [1.0 / (jnp.sqrt(jnp.maximum(v, 0.0)) + 1e-8) for v in vs]
    norm = r[0] + r[1] + r[2]
    S = (jnp.where(iota == ix[0], r[0], 0.0)
         + jnp.where(iota == ix[1], r[1], 0.0)
         + jnp.where(iota == ix[2], r[2], 0.0)) / norm

    interp = jax.lax.dot(kf_ref[0], S, preferred_element_type=jnp.float32)
    w0 = w0_ref[...]
    x = (jax.lax.dot(w0[:, :_C2], interp, preferred_element_type=jnp.float32)
         + jax.lax.dot(w0[:, _C2:], uf_ref[0],
                       preferred_element_type=jnp.float32))
    x_ref[0] = x

    @pl.when((pl.program_id(0) == 0) & (pl.program_id(1) == 0))
    def _init():
        sum_ref[...] = jnp.zeros_like(sum_ref)
        ssq_ref[...] = jnp.zeros_like(ssq_ref)

    sum_ref[...] += jnp.sum(x, axis=1, keepdims=True)
    ssq_ref[...] += jnp.sum(x * x, axis=1, keepdims=True)


def _bn_relu_kernel(x_ref, sum_ref, ssq_ref, g_ref, b_ref, o_ref):
    cnt = jnp.float32(_B * _N)
    mean = sum_ref[...] / cnt
    var = ssq_ref[...] / cnt - mean * mean
    scale = g_ref[...] * jax.lax.rsqrt(var + 1e-5)
    shift = b_ref[...] - mean * scale
    o_ref[0] = jnp.maximum(x_ref[0] * scale + shift, 0.0)


def kernel(unknown, known, unknow_feats, known_feats, W0, gamma0, beta0):
    uT = jnp.transpose(unknown, (0, 2, 1))
    grid = (_B, _NB)
    x, s, ss = pl.pallas_call(
        _fp_fwd_kernel,
        grid=grid,
        in_specs=[
            pl.BlockSpec((1, 3, _NT), lambda b, n: (b, 0, n)),
            pl.BlockSpec((1, _M, 3), lambda b, n: (b, 0, 0)),
            pl.BlockSpec((1, _C1, _NT), lambda b, n: (b, 0, n)),
            pl.BlockSpec((1, _C2, _M), lambda b, n: (b, 0, 0)),
            pl.BlockSpec((_COUT, _C1 + _C2), lambda b, n: (0, 0)),
        ],
        out_specs=[
            pl.BlockSpec((1, _COUT, _NT), lambda b, n: (b, 0, n)),
            pl.BlockSpec((_COUT, 1), lambda b, n: (0, 0)),
            pl.BlockSpec((_COUT, 1), lambda b, n: (0, 0)),
        ],
        out_shape=[
            jax.ShapeDtypeStruct((_B, _COUT, _N), jnp.float32),
            jax.ShapeDtypeStruct((_COUT, 1), jnp.float32),
            jax.ShapeDtypeStruct((_COUT, 1), jnp.float32),
        ],
    )(uT, known, unknow_feats, known_feats, W0)

    out = pl.pallas_call(
        _bn_relu_kernel,
        grid=grid,
        in_specs=[
            pl.BlockSpec((1, _COUT, _NT), lambda b, n: (b, 0, n)),
            pl.BlockSpec((_COUT, 1), lambda b, n: (0, 0)),
            pl.BlockSpec((_COUT, 1), lambda b, n: (0, 0)),
            pl.BlockSpec((_COUT, 1), lambda b, n: (0, 0)),
            pl.BlockSpec((_COUT, 1), lambda b, n: (0, 0)),
        ],
        out_specs=pl.BlockSpec((1, _COUT, _NT), lambda b, n: (b, 0, n)),
        out_shape=jax.ShapeDtypeStruct((_B, _COUT, _N), jnp.float32),
    )(x, s, ss, gamma0.reshape(_COUT, 1), beta0.reshape(_COUT, 1))
    return out


# speedup vs baseline: 23.5773x; 1.2549x over previous
import jax
import jax.numpy as jnp
from jax.experimental import pallas as pl

_B, _N, _M, _C1, _C2 = 4, 8192, 1024, 64, 128
_COUT = 128
_NT = 256
_NB = _N // _NT


def _fp_fwd_kernel(uT_ref, known_ref, uf_ref, kf_ref, w0_ref,
                   x_ref, sum_ref, ssq_ref):
    uT = uT_ref[0]
    kn = known_ref[0]

    cross = jax.lax.dot(kn.astype(jnp.bfloat16), uT.astype(jnp.bfloat16),
                        preferred_element_type=jnp.float32)
    k2 = jnp.sum(kn * kn, axis=1, keepdims=True)
    u2 = jnp.sum(uT * uT, axis=0, keepdims=True)
    d2 = (u2 + k2) - 2.0 * cross

    inf = jnp.float32(jnp.inf)
    v1 = jnp.min(d2, axis=0, keepdims=True)
    w2 = jnp.where(d2 == v1, inf, d2)
    v2 = jnp.min(w2, axis=0, keepdims=True)
    w3 = jnp.where(w2 == v2, inf, w2)
    v3 = jnp.min(w3, axis=0, keepdims=True)

    r1, r2, r3 = [1.0 / (jnp.sqrt(jnp.maximum(v, 0.0)) + 1e-8)
                  for v in (v1, v2, v3)]
    norm = r1 + r2 + r3
    S = (jnp.where(d2 == v1, r1, 0.0)
         + jnp.where(d2 == v2, r2, 0.0)
         + jnp.where(d2 == v3, r3, 0.0)) / norm

    interp = jax.lax.dot(kf_ref[0], S, preferred_element_type=jnp.float32)
    w0 = w0_ref[...]
    x = (jax.lax.dot(w0[:, :_C2], interp, preferred_element_type=jnp.float32)
         + jax.lax.dot(w0[:, _C2:], uf_ref[0],
                       preferred_element_type=jnp.float32))
    x_ref[0] = x

    @pl.when((pl.program_id(0) == 0) & (pl.program_id(1) == 0))
    def _init():
        sum_ref[...] = jnp.zeros_like(sum_ref)
        ssq_ref[...] = jnp.zeros_like(ssq_ref)

    sum_ref[...] += jnp.sum(x, axis=1, keepdims=True)
    ssq_ref[...] += jnp.sum(x * x, axis=1, keepdims=True)


def _bn_relu_kernel(x_ref, sum_ref, ssq_ref, g_ref, b_ref, o_ref):
    cnt = jnp.float32(_B * _N)
    mean = sum_ref[...] / cnt
    var = ssq_ref[...] / cnt - mean * mean
    scale = g_ref[...] * jax.lax.rsqrt(var + 1e-5)
    shift = b_ref[...] - mean * scale
    o_ref[0] = jnp.maximum(x_ref[0] * scale + shift, 0.0)


def kernel(unknown, known, unknow_feats, known_feats, W0, gamma0, beta0):
    uT = jnp.transpose(unknown, (0, 2, 1))
    grid = (_B, _NB)
    x, s, ss = pl.pallas_call(
        _fp_fwd_kernel,
        grid=grid,
        in_specs=[
            pl.BlockSpec((1, 3, _NT), lambda b, n: (b, 0, n)),
            pl.BlockSpec((1, _M, 3), lambda b, n: (b, 0, 0)),
            pl.BlockSpec((1, _C1, _NT), lambda b, n: (b, 0, n)),
            pl.BlockSpec((1, _C2, _M), lambda b, n: (b, 0, 0)),
            pl.BlockSpec((_COUT, _C1 + _C2), lambda b, n: (0, 0)),
        ],
        out_specs=[
            pl.BlockSpec((1, _COUT, _NT), lambda b, n: (b, 0, n)),
            pl.BlockSpec((_COUT, 1), lambda b, n: (0, 0)),
            pl.BlockSpec((_COUT, 1), lambda b, n: (0, 0)),
        ],
        out_shape=[
            jax.ShapeDtypeStruct((_B, _COUT, _N), jnp.float32),
            jax.ShapeDtypeStruct((_COUT, 1), jnp.float32),
            jax.ShapeDtypeStruct((_COUT, 1), jnp.float32),
        ],
    )(uT, known, unknow_feats, known_feats, W0)

    out = pl.pallas_call(
        _bn_relu_kernel,
        grid=grid,
        in_specs=[
            pl.BlockSpec((1, _COUT, _NT), lambda b, n: (b, 0, n)),
            pl.BlockSpec((_COUT, 1), lambda b, n: (0, 0)),
            pl.BlockSpec((_COUT, 1), lambda b, n: (0, 0)),
            pl.BlockSpec((_COUT, 1), lambda b, n: (0, 0)),
            pl.BlockSpec((_COUT, 1), lambda b, n: (0, 0)),
        ],
        out_specs=pl.BlockSpec((1, _COUT, _NT), lambda b, n: (b, 0, n)),
        out_shape=jax.ShapeDtypeStruct((_B, _COUT, _N), jnp.float32),
    )(x, s, ss, gamma0.reshape(_COUT, 1), beta0.reshape(_COUT, 1))
    return out


# speedup vs baseline: 36.8240x; 1.5618x over previous
import jax
import jax.numpy as jnp
from jax.experimental import pallas as pl

_B, _N, _M, _C1, _C2 = 4, 8192, 1024, 64, 128
_COUT = 128
_NT = 512
_NB = _N // _NT


def _fp_fwd_kernel(uT_ref, known_ref, uf_ref, kf_ref, w0_ref,
                   x_ref, sum_ref, ssq_ref):
    uT = uT_ref[0]
    kn = known_ref[0]

    cross = jax.lax.dot(kn.astype(jnp.bfloat16), uT.astype(jnp.bfloat16),
                        preferred_element_type=jnp.float32)
    k2 = jnp.sum(kn * kn, axis=1, keepdims=True)
    u2 = jnp.sum(uT * uT, axis=0, keepdims=True)
    d2 = (u2 + k2) - 2.0 * cross

    inf = jnp.float32(jnp.inf)
    v1 = jnp.min(d2, axis=0, keepdims=True)
    c1 = d2 == v1
    w2 = jnp.where(c1, inf, d2)
    v2 = jnp.min(w2, axis=0, keepdims=True)
    c2 = w2 == v2
    w3 = jnp.where(c2, inf, w2)
    v3 = jnp.min(w3, axis=0, keepdims=True)
    c3 = w3 == v3

    r1, r2, r3 = [1.0 / (jnp.sqrt(jnp.maximum(v, 0.0)) + 1e-8)
                  for v in (v1, v2, v3)]
    inv_norm = 1.0 / (r1 + r2 + r3)
    S = jnp.where(c1, r1 * inv_norm,
                  jnp.where(c2, r2 * inv_norm,
                            jnp.where(c3, r3 * inv_norm, 0.0)))

    interp = jax.lax.dot(kf_ref[0], S, preferred_element_type=jnp.float32)
    w0 = w0_ref[...]
    x = (jax.lax.dot(w0[:, :_C2], interp, preferred_element_type=jnp.float32)
         + jax.lax.dot(w0[:, _C2:], uf_ref[0],
                       preferred_element_type=jnp.float32))
    x_ref[0] = x

    @pl.when((pl.program_id(0) == 0) & (pl.program_id(1) == 0))
    def _init():
        sum_ref[...] = jnp.zeros_like(sum_ref)
        ssq_ref[...] = jnp.zeros_like(ssq_ref)

    sum_ref[...] += jnp.sum(x, axis=1, keepdims=True)
    ssq_ref[...] += jnp.sum(x * x, axis=1, keepdims=True)


def _bn_relu_kernel(x_ref, sum_ref, ssq_ref, g_ref, b_ref, o_ref):
    cnt = jnp.float32(_B * _N)
    mean = sum_ref[...] / cnt
    var = ssq_ref[...] / cnt - mean * mean
    scale = g_ref[...] * jax.lax.rsqrt(var + 1e-5)
    shift = b_ref[...] - mean * scale
    o_ref[0] = jnp.maximum(x_ref[0] * scale + shift, 0.0)


def kernel(unknown, known, unknow_feats, known_feats, W0, gamma0, beta0):
    uT = jnp.transpose(unknown, (0, 2, 1))
    grid = (_B, _NB)
    x, s, ss = pl.pallas_call(
        _fp_fwd_kernel,
        grid=grid,
        in_specs=[
            pl.BlockSpec((1, 3, _NT), lambda b, n: (b, 0, n)),
            pl.BlockSpec((1, _M, 3), lambda b, n: (b, 0, 0)),
            pl.BlockSpec((1, _C1, _NT), lambda b, n: (b, 0, n)),
            pl.BlockSpec((1, _C2, _M), lambda b, n: (b, 0, 0)),
            pl.BlockSpec((_COUT, _C1 + _C2), lambda b, n: (0, 0)),
        ],
        out_specs=[
            pl.BlockSpec((1, _COUT, _NT), lambda b, n: (b, 0, n)),
            pl.BlockSpec((_COUT, 1), lambda b, n: (0, 0)),
            pl.BlockSpec((_COUT, 1), lambda b, n: (0, 0)),
        ],
        out_shape=[
            jax.ShapeDtypeStruct((_B, _COUT, _N), jnp.float32),
            jax.ShapeDtypeStruct((_COUT, 1), jnp.float32),
            jax.ShapeDtypeStruct((_COUT, 1), jnp.float32),
        ],
    )(uT, known, unknow_feats, known_feats, W0)

    out = pl.pallas_call(
        _bn_relu_kernel,
        grid=grid,
        in_specs=[
            pl.BlockSpec((1, _COUT, _NT), lambda b, n: (b, 0, n)),
            pl.BlockSpec((_COUT, 1), lambda b, n: (0, 0)),
            pl.BlockSpec((_COUT, 1), lambda b, n: (0, 0)),
            pl.BlockSpec((_COUT, 1), lambda b, n: (0, 0)),
            pl.BlockSpec((_COUT, 1), lambda b, n: (0, 0)),
        ],
        out_specs=pl.BlockSpec((1, _COUT, _NT), lambda b, n: (b, 0, n)),
        out_shape=jax.ShapeDtypeStruct((_B, _COUT, _N), jnp.float32),
    )(x, s, ss, gamma0.reshape(_COUT, 1), beta0.reshape(_COUT, 1))
    return out
